# baseline (speedup 1.0000x reference)
import functools

import jax
import jax.numpy as jnp
from jax import lax
from jax.experimental import pallas as pl
from jax.experimental.pallas import tpu as pltpu
from jax.experimental.pallas import tpu_sc as plsc

N = 10000
E = 320000
D = 128
T_STEPS = 8

NC = 2
NS = 16
NW = NC * NS
CH = 40
NCHUNK = E // CH
FULL_J = NCHUNK // NW
NTAIL = NCHUNK - FULL_J * NW
N_ACC = N
RSL = 624


@functools.lru_cache(maxsize=None)
def _get_mesh():
    return plsc.VectorSubcoreMesh(
        core_axis_name="c", subcore_axis_name="s",
        num_cores=NC, num_subcores=NS)


NCHAIN = 8
TSTEPS_MP = FULL_J // NCHAIN


@functools.lru_cache(maxsize=None)
def _get_mp():
    return functools.partial(
        pl.kernel,
        out_type=jax.ShapeDtypeStruct((NC, N, D), jnp.float32),
        mesh=_get_mesh(),
        scratch_types=(
            [pltpu.VMEM((CH,), jnp.int32)] * NCHAIN
            + [pltpu.VMEM((CH,), jnp.int32)] * NCHAIN
            + [pltpu.VMEM((NCHAIN, CH, D), jnp.float32),
               pltpu.VMEM_SHARED((N_ACC, D), jnp.float32)]
            + [pltpu.SemaphoreType.DMA] * (3 * NCHAIN)
        ),
    )(_mp_body)


def _mp_body(src_hbm, dst_hbm, g_hbm, zeros_hbm, out_hbm, *rest):
    sv = rest[:NCHAIN]
    dv = rest[NCHAIN:2 * NCHAIN]
    bufs, acc = rest[2 * NCHAIN], rest[2 * NCHAIN + 1]
    sems = rest[2 * NCHAIN + 2:]
    isem = sems[:NCHAIN]
    gsem = sems[NCHAIN:2 * NCHAIN]
    ssem = sems[2 * NCHAIN:]
    c = lax.axis_index("c")
    s = lax.axis_index("s")
    wid = s * NC + c

    r0 = s * RSL
    pltpu.sync_copy(zeros_hbm.at[pl.ds(r0, RSL)], acc.at[pl.ds(r0, RSL)])

    @pl.when(s == 0)
    def _():
        tail = NS * RSL
        pltpu.sync_copy(zeros_hbm.at[pl.ds(tail, N - tail)],
                        acc.at[pl.ds(tail, N - tail)])

    plsc.subcore_barrier()

    def body(t, carry):
        idd = []
        for k in range(NCHAIN):
            j = t * NCHAIN + k
            base = (wid + j * NW) * CH

            @pl.when(t > 0)
            def _(k=k, j=j):
                pltpu.make_async_copy(
                    bufs.at[k], acc.at[dv[k]], ssem[k]).wait()

            idd.append((
                pltpu.async_copy(src_hbm.at[pl.ds(base, CH)], sv[k], isem[k]),
                pltpu.async_copy(dst_hbm.at[pl.ds(base, CH)], dv[k], isem[k]),
            ))
        gd = []
        for k in range(NCHAIN):
            idd[k][0].wait()
            idd[k][1].wait()
            gd.append(pltpu.async_copy(g_hbm.at[sv[k]], bufs.at[k], gsem[k]))
        for k in range(NCHAIN):
            gd[k].wait()
            pltpu.async_copy(bufs.at[k], acc.at[dv[k]], ssem[k], add=True)
        return carry

    lax.fori_loop(0, TSTEPS_MP, body, 0)

    def _tail_chunk(base):
        pltpu.make_async_copy(bufs.at[0], acc.at[dv[0]], ssem[0]).wait()
        i0 = pltpu.async_copy(src_hbm.at[pl.ds(base, CH)], sv[0], isem[0])
        i1 = pltpu.async_copy(dst_hbm.at[pl.ds(base, CH)], dv[0], isem[0])
        i0.wait()
        i1.wait()
        pltpu.async_copy(g_hbm.at[sv[0]], bufs.at[0], gsem[0]).wait()
        pltpu.async_copy(bufs.at[0], acc.at[dv[0]], ssem[0], add=True)

    for j in range(TSTEPS_MP * NCHAIN, FULL_J):
        _tail_chunk((wid + j * NW) * CH)

    @pl.when(wid < NTAIL)
    def _():
        _tail_chunk((wid + FULL_J * NW) * CH)

    for k in range(NCHAIN):
        pltpu.make_async_copy(bufs.at[k], acc.at[dv[k]], ssem[k]).wait()

    plsc.subcore_barrier()
    pltpu.sync_copy(acc.at[pl.ds(r0, RSL)], out_hbm.at[c, pl.ds(r0, RSL)])

    @pl.when(s == 0)
    def _():
        tail = NS * RSL
        pltpu.sync_copy(acc.at[pl.ds(tail, N - tail)],
                        out_hbm.at[c, pl.ds(tail, N - tail)])


R = 2000


def _row_spec(r, w):
    return pl.BlockSpec((r, w), lambda i: (i, 0))


def _full_spec(shape):
    return pl.BlockSpec(shape, lambda i: (0,) * len(shape))


def _silu(y):
    return y * jax.nn.sigmoid(y)


def _mm_emb_body(h, w, b, o):
    o[...] = jnp.dot(h[...], w[...], preferred_element_type=jnp.float32) + b[...]


def _mm_emb(h, w, b):
    return pl.pallas_call(
        _mm_emb_body,
        grid=(h.shape[0] // R,),
        in_specs=[_row_spec(R, D), _full_spec((D, D)), _full_spec((1, D))],
        out_specs=_row_spec(R, D),
        out_shape=jax.ShapeDtypeStruct((h.shape[0], D), jnp.float32),
    )(h, w, b)


def _inv_body(d0, d1, o):
    o[...] = 1.0 / (d0[:, :1] + d1[:, :1] + 1.0)


def _inv_deg(deg):
    return pl.pallas_call(
        _inv_body,
        grid=(N // R,),
        in_specs=[_row_spec(R, D), _row_spec(R, D)],
        out_specs=_row_spec(R, 1),
        out_shape=jax.ShapeDtypeStruct((N, 1), jnp.float32),
    )(deg[0], deg[1])


def _gcn_k(a, w, b, inv, y):
    t = (a[0][...] + a[1][...] + y[...]) * inv[...]
    return _silu(jnp.dot(t, w[...], preferred_element_type=jnp.float32) + b[...])


def _enc_body(a0, a1, y, inv, w, b, o):
    o[...] = _gcn_k((a0, a1), w, b, inv, y)


def _enc(agg, y, inv, w, b):
    return pl.pallas_call(
        _enc_body,
        grid=(N // R,),
        in_specs=[_row_spec(R, D)] * 3 + [_row_spec(R, 1)]
        + [_full_spec((D, D)), _full_spec((1, D))],
        out_specs=_row_spec(R, D),
        out_shape=jax.ShapeDtypeStruct((N, D), jnp.float32),
    )(agg[0], agg[1], y, inv, w, b)


def _stage_body(c_next, a0, a1, y, inv, w, b, hc, k_o, y_o):
    k = _gcn_k((a0, a1), w, b, inv, y)
    k_o[...] = k
    y_o[...] = hc[...] + c_next * k


def _rk_stage(c_next, agg, y, inv, w, b, hc):
    return pl.pallas_call(
        functools.partial(_stage_body, c_next),
        grid=(N // R,),
        in_specs=[_row_spec(R, D)] * 3 + [_row_spec(R, 1)]
        + [_full_spec((D, D)), _full_spec((1, D))] + [_row_spec(R, D)],
        out_specs=[_row_spec(R, D)] * 2,
        out_shape=[jax.ShapeDtypeStruct((N, D), jnp.float32)] * 2,
    )(agg[0], agg[1], y, inv, w, b, hc)


def _final_body(dt6, a0, a1, y, inv, w, b, hc, k1, k2, k3, o):
    k4 = _gcn_k((a0, a1), w, b, inv, y)
    o[...] = hc[...] + dt6 * (k1[...] + 2.0 * (k2[...] + k3[...]) + k4)


def _rk_final(dt6, agg, y, inv, w, b, hc, k1, k2, k3):
    return pl.pallas_call(
        functools.partial(_final_body, dt6),
        grid=(N // R,),
        in_specs=[_row_spec(R, D)] * 3 + [_row_spec(R, 1)]
        + [_full_spec((D, D)), _full_spec((1, D))] + [_row_spec(R, D)] * 4,
        out_specs=_row_spec(R, D),
        out_shape=jax.ShapeDtypeStruct((N, D), jnp.float32),
    )(agg[0], agg[1], y, inv, w, b, hc, k1, k2, k3)


def _dec_body(h, w1, b1, w2, b2, o):
    t = _silu(jnp.dot(h[...], w1[...], preferred_element_type=jnp.float32) + b1[...])
    o[...] = jnp.dot(t, w2[...], preferred_element_type=jnp.float32) + b2[...]


def _decode(hs, w1, b1, w2, b2):
    m = hs.shape[0]
    return pl.pallas_call(
        _dec_body,
        grid=(m // R,),
        in_specs=[_row_spec(R, D), _full_spec((D, D)), _full_spec((1, D)),
                  _full_spec((D, D)), _full_spec((1, D))],
        out_specs=_row_spec(R, D),
        out_shape=jax.ShapeDtypeStruct((m, D), jnp.float32),
    )(hs, w1, b1, w2, b2)


def kernel(x, h, edge_index, edge_fea, W_emb, b_emb, W_enc0, b_enc0,
           W_enc1, b_enc1, W_ode, b_ode, W_d1, b_d1, W_d2, b_d2):
    src = edge_index[0]
    dst = edge_index[1]

    zeros_nd = jnp.zeros((N, D), jnp.float32)
    ones_nd = jnp.ones((N, D), jnp.float32)

    mp_call = _get_mp()
    deg = mp_call(src, dst, ones_nd, zeros_nd)
    inv = _inv_deg(deg)
    h, inv = lax.optimization_barrier((h, inv))

    b_emb2 = b_emb.reshape(1, D)
    b_enc0_2 = b_enc0.reshape(1, D)
    b_enc1_2 = b_enc1.reshape(1, D)
    b_ode2 = b_ode.reshape(1, D)
    b_d1_2 = b_d1.reshape(1, D)
    W_d2p = jnp.zeros((D, D), jnp.float32).at[:, :3].set(W_d2)
    b_d2p = jnp.zeros((1, D), jnp.float32).at[0, :3].set(b_d2)

    def mp(g):
        return mp_call(src, dst, g, zeros_nd)

    h0 = _mm_emb(h, W_emb, b_emb2)
    h1 = _enc(mp(h0), h0, inv, W_enc0, b_enc0_2)
    h2 = _enc(mp(h1), h1, inv, W_enc1, b_enc1_2)

    dt = 1.0 / (T_STEPS - 1)
    sol = [h2]
    hc = h2
    for _ in range(T_STEPS - 1):
        k1, y2 = _rk_stage(0.5 * dt, mp(hc), hc, inv, W_ode, b_ode2, hc)
        k2, y3 = _rk_stage(0.5 * dt, mp(y2), y2, inv, W_ode, b_ode2, hc)
        k3, y4 = _rk_stage(dt, mp(y3), y3, inv, W_ode, b_ode2, hc)
        hc = _rk_final(dt / 6.0, mp(y4), y4, inv, W_ode, b_ode2, hc, k1, k2, k3)
        sol.append(hc)

    hs = jnp.concatenate(sol, axis=0)
    xd = _decode(hs, W_d1, b_d1_2, W_d2p, b_d2p)
    x_final = xd[:, :3]
    dummy_v = jnp.zeros_like(x_final)
    kld = jnp.float32(0.0)
    return (x_final, dummy_v, sol[-1], kld)


# speedup vs baseline: 1.0167x; 1.0167x over previous
import functools

import jax
import jax.numpy as jnp
from jax import lax
from jax.experimental import pallas as pl
from jax.experimental.pallas import tpu as pltpu
from jax.experimental.pallas import tpu_sc as plsc

N = 10000
E = 320000
D = 128
T_STEPS = 8

NC = 2
NS = 16
NW = NC * NS
CH = 64
NCHUNK = E // CH
FULL_J = NCHUNK // NW
NTAIL = NCHUNK - FULL_J * NW
N_ACC = N
RSL = 624


@functools.lru_cache(maxsize=None)
def _get_mesh():
    return plsc.VectorSubcoreMesh(
        core_axis_name="c", subcore_axis_name="s",
        num_cores=NC, num_subcores=NS)


NCHAIN = 6
TSTEPS_MP = FULL_J // NCHAIN


@functools.lru_cache(maxsize=None)
def _get_mp():
    return functools.partial(
        pl.kernel,
        out_type=jax.ShapeDtypeStruct((NC, N, D), jnp.float32),
        mesh=_get_mesh(),
        scratch_types=(
            [pltpu.VMEM((CH,), jnp.int32)] * NCHAIN
            + [pltpu.VMEM((CH,), jnp.int32)] * NCHAIN
            + [pltpu.VMEM((NCHAIN, CH, D), jnp.float32),
               pltpu.VMEM_SHARED((N_ACC, D), jnp.float32)]
            + [pltpu.SemaphoreType.DMA] * (3 * NCHAIN)
        ),
    )(_mp_body)


def _mp_body(src_hbm, dst_hbm, g_hbm, zeros_hbm, out_hbm, *rest):
    sv = rest[:NCHAIN]
    dv = rest[NCHAIN:2 * NCHAIN]
    bufs, acc = rest[2 * NCHAIN], rest[2 * NCHAIN + 1]
    sems = rest[2 * NCHAIN + 2:]
    isem = sems[:NCHAIN]
    gsem = sems[NCHAIN:2 * NCHAIN]
    ssem = sems[2 * NCHAIN:]
    c = lax.axis_index("c")
    s = lax.axis_index("s")
    wid = s * NC + c

    r0 = s * RSL
    pltpu.sync_copy(zeros_hbm.at[pl.ds(r0, RSL)], acc.at[pl.ds(r0, RSL)])

    @pl.when(s == 0)
    def _():
        tail = NS * RSL
        pltpu.sync_copy(zeros_hbm.at[pl.ds(tail, N - tail)],
                        acc.at[pl.ds(tail, N - tail)])

    plsc.subcore_barrier()

    def body(t, carry):
        idd = []
        for k in range(NCHAIN):
            j = t * NCHAIN + k
            base = (wid + j * NW) * CH

            @pl.when(t > 0)
            def _(k=k, j=j):
                pltpu.make_async_copy(
                    bufs.at[k], acc.at[dv[k]], ssem[k]).wait()

            idd.append((
                pltpu.async_copy(src_hbm.at[pl.ds(base, CH)], sv[k], isem[k]),
                pltpu.async_copy(dst_hbm.at[pl.ds(base, CH)], dv[k], isem[k]),
            ))
        gd = []
        for k in range(NCHAIN):
            idd[k][0].wait()
            idd[k][1].wait()
            gd.append(pltpu.async_copy(g_hbm.at[sv[k]], bufs.at[k], gsem[k]))
        for k in range(NCHAIN):
            gd[k].wait()
            pltpu.async_copy(bufs.at[k], acc.at[dv[k]], ssem[k], add=True)
        return carry

    lax.fori_loop(0, TSTEPS_MP, body, 0)

    def _tail_chunk(base):
        pltpu.make_async_copy(bufs.at[0], acc.at[dv[0]], ssem[0]).wait()
        i0 = pltpu.async_copy(src_hbm.at[pl.ds(base, CH)], sv[0], isem[0])
        i1 = pltpu.async_copy(dst_hbm.at[pl.ds(base, CH)], dv[0], isem[0])
        i0.wait()
        i1.wait()
        pltpu.async_copy(g_hbm.at[sv[0]], bufs.at[0], gsem[0]).wait()
        pltpu.async_copy(bufs.at[0], acc.at[dv[0]], ssem[0], add=True)

    for j in range(TSTEPS_MP * NCHAIN, FULL_J):
        _tail_chunk((wid + j * NW) * CH)

    @pl.when(wid < NTAIL)
    def _():
        _tail_chunk((wid + FULL_J * NW) * CH)

    for k in range(NCHAIN):
        pltpu.make_async_copy(bufs.at[k], acc.at[dv[k]], ssem[k]).wait()

    plsc.subcore_barrier()
    pltpu.sync_copy(acc.at[pl.ds(r0, RSL)], out_hbm.at[c, pl.ds(r0, RSL)])

    @pl.when(s == 0)
    def _():
        tail = NS * RSL
        pltpu.sync_copy(acc.at[pl.ds(tail, N - tail)],
                        out_hbm.at[c, pl.ds(tail, N - tail)])


@functools.lru_cache(maxsize=None)
def _get_deg():
    return functools.partial(
        pl.kernel,
        out_type=jax.ShapeDtypeStruct((NC, N, D), jnp.float32),
        mesh=_get_mesh(),
        scratch_types=(
            [pltpu.VMEM((CH,), jnp.int32)] * NCHAIN
            + [pltpu.VMEM((CH, D), jnp.float32),
               pltpu.VMEM_SHARED((N_ACC, D), jnp.float32)]
            + [pltpu.SemaphoreType.DMA] * (2 * NCHAIN)
        ),
    )(_deg_body)


def _deg_body(dst_hbm, ones_hbm, zeros_hbm, out_hbm, *rest):
    dv = rest[:NCHAIN]
    ones_buf, acc = rest[NCHAIN], rest[NCHAIN + 1]
    sems = rest[NCHAIN + 2:]
    isem = sems[:NCHAIN]
    ssem = sems[NCHAIN:]
    c = lax.axis_index("c")
    s = lax.axis_index("s")
    wid = s * NC + c

    r0 = s * RSL
    pltpu.sync_copy(zeros_hbm.at[pl.ds(r0, RSL)], acc.at[pl.ds(r0, RSL)])

    @pl.when(s == 0)
    def _():
        tail = NS * RSL
        pltpu.sync_copy(zeros_hbm.at[pl.ds(tail, N - tail)],
                        acc.at[pl.ds(tail, N - tail)])

    pltpu.sync_copy(ones_hbm.at[pl.ds(0, CH)], ones_buf)
    plsc.subcore_barrier()

    def body(t, carry):
        idd = []
        for k in range(NCHAIN):
            j = t * NCHAIN + k
            base = (wid + j * NW) * CH

            @pl.when(t > 0)
            def _(k=k):
                pltpu.make_async_copy(ones_buf, acc.at[dv[k]], ssem[k]).wait()

            idd.append(
                pltpu.async_copy(dst_hbm.at[pl.ds(base, CH)], dv[k], isem[k]))
        for k in range(NCHAIN):
            idd[k].wait()
            pltpu.async_copy(ones_buf, acc.at[dv[k]], ssem[k], add=True)
        return carry

    lax.fori_loop(0, TSTEPS_MP, body, 0)

    for j in range(TSTEPS_MP * NCHAIN, FULL_J):
        base = (wid + j * NW) * CH
        pltpu.make_async_copy(ones_buf, acc.at[dv[0]], ssem[0]).wait()
        pltpu.sync_copy(dst_hbm.at[pl.ds(base, CH)], dv[0])
        pltpu.async_copy(ones_buf, acc.at[dv[0]], ssem[0], add=True)

    @pl.when(wid < NTAIL)
    def _():
        base = (wid + FULL_J * NW) * CH
        pltpu.make_async_copy(ones_buf, acc.at[dv[0]], ssem[0]).wait()
        pltpu.sync_copy(dst_hbm.at[pl.ds(base, CH)], dv[0])
        pltpu.async_copy(ones_buf, acc.at[dv[0]], ssem[0], add=True)

    for k in range(NCHAIN):
        pltpu.make_async_copy(ones_buf, acc.at[dv[k]], ssem[k]).wait()

    plsc.subcore_barrier()
    pltpu.sync_copy(acc.at[pl.ds(r0, RSL)], out_hbm.at[c, pl.ds(r0, RSL)])

    @pl.when(s == 0)
    def _():
        tail = NS * RSL
        pltpu.sync_copy(acc.at[pl.ds(tail, N - tail)],
                        out_hbm.at[c, pl.ds(tail, N - tail)])


R = 2000


def _row_spec(r, w):
    return pl.BlockSpec((r, w), lambda i: (i, 0))


def _full_spec(shape):
    return pl.BlockSpec(shape, lambda i: (0,) * len(shape))


def _silu(y):
    return y * jax.nn.sigmoid(y)


def _mm_emb_body(h, w, b, o):
    o[...] = jnp.dot(h[...], w[...], preferred_element_type=jnp.float32) + b[...]


def _mm_emb(h, w, b):
    return pl.pallas_call(
        _mm_emb_body,
        grid=(h.shape[0] // R,),
        in_specs=[_row_spec(R, D), _full_spec((D, D)), _full_spec((1, D))],
        out_specs=_row_spec(R, D),
        out_shape=jax.ShapeDtypeStruct((h.shape[0], D), jnp.float32),
    )(h, w, b)


def _inv_body(d0, d1, o):
    o[...] = 1.0 / (d0[:, :1] + d1[:, :1] + 1.0)


def _inv_deg(deg):
    return pl.pallas_call(
        _inv_body,
        grid=(N // R,),
        in_specs=[_row_spec(R, D), _row_spec(R, D)],
        out_specs=_row_spec(R, 1),
        out_shape=jax.ShapeDtypeStruct((N, 1), jnp.float32),
    )(deg[0], deg[1])


def _gcn_k(a, w, b, inv, y):
    t = (a[0][...] + a[1][...] + y[...]) * inv[...]
    return _silu(jnp.dot(t, w[...], preferred_element_type=jnp.float32) + b[...])


def _enc_body(a0, a1, y, inv, w, b, o):
    o[...] = _gcn_k((a0, a1), w, b, inv, y)


def _enc(agg, y, inv, w, b):
    return pl.pallas_call(
        _enc_body,
        grid=(N // R,),
        in_specs=[_row_spec(R, D)] * 3 + [_row_spec(R, 1)]
        + [_full_spec((D, D)), _full_spec((1, D))],
        out_specs=_row_spec(R, D),
        out_shape=jax.ShapeDtypeStruct((N, D), jnp.float32),
    )(agg[0], agg[1], y, inv, w, b)


def _stage_body(c_next, a0, a1, y, inv, w, b, hc, k_o, y_o):
    k = _gcn_k((a0, a1), w, b, inv, y)
    k_o[...] = k
    y_o[...] = hc[...] + c_next * k


def _rk_stage(c_next, agg, y, inv, w, b, hc):
    return pl.pallas_call(
        functools.partial(_stage_body, c_next),
        grid=(N // R,),
        in_specs=[_row_spec(R, D)] * 3 + [_row_spec(R, 1)]
        + [_full_spec((D, D)), _full_spec((1, D))] + [_row_spec(R, D)],
        out_specs=[_row_spec(R, D)] * 2,
        out_shape=[jax.ShapeDtypeStruct((N, D), jnp.float32)] * 2,
    )(agg[0], agg[1], y, inv, w, b, hc)


def _final_body(dt6, a0, a1, y, inv, w, b, hc, k1, k2, k3, o):
    k4 = _gcn_k((a0, a1), w, b, inv, y)
    o[...] = hc[...] + dt6 * (k1[...] + 2.0 * (k2[...] + k3[...]) + k4)


def _rk_final(dt6, agg, y, inv, w, b, hc, k1, k2, k3):
    return pl.pallas_call(
        functools.partial(_final_body, dt6),
        grid=(N // R,),
        in_specs=[_row_spec(R, D)] * 3 + [_row_spec(R, 1)]
        + [_full_spec((D, D)), _full_spec((1, D))] + [_row_spec(R, D)] * 4,
        out_specs=_row_spec(R, D),
        out_shape=jax.ShapeDtypeStruct((N, D), jnp.float32),
    )(agg[0], agg[1], y, inv, w, b, hc, k1, k2, k3)


def _dec_body(h, w1, b1, w2, b2, o):
    t = _silu(jnp.dot(h[...], w1[...], preferred_element_type=jnp.float32) + b1[...])
    o[...] = jnp.dot(t, w2[...], preferred_element_type=jnp.float32) + b2[...]


def _decode(hs, w1, b1, w2, b2):
    m = hs.shape[0]
    return pl.pallas_call(
        _dec_body,
        grid=(m // R,),
        in_specs=[_row_spec(R, D), _full_spec((D, D)), _full_spec((1, D)),
                  _full_spec((D, D)), _full_spec((1, D))],
        out_specs=_row_spec(R, D),
        out_shape=jax.ShapeDtypeStruct((m, D), jnp.float32),
    )(hs, w1, b1, w2, b2)


def kernel(x, h, edge_index, edge_fea, W_emb, b_emb, W_enc0, b_enc0,
           W_enc1, b_enc1, W_ode, b_ode, W_d1, b_d1, W_d2, b_d2):
    src = edge_index[0]
    dst = edge_index[1]

    zeros_nd = jnp.zeros((N, D), jnp.float32)
    ones_nd = jnp.ones((N, D), jnp.float32)

    mp_call = _get_mp()
    deg = _get_deg()(dst, ones_nd, zeros_nd)
    inv = _inv_deg(deg)
    h, inv = lax.optimization_barrier((h, inv))

    b_emb2 = b_emb.reshape(1, D)
    b_enc0_2 = b_enc0.reshape(1, D)
    b_enc1_2 = b_enc1.reshape(1, D)
    b_ode2 = b_ode.reshape(1, D)
    b_d1_2 = b_d1.reshape(1, D)
    W_d2p = jnp.zeros((D, D), jnp.float32).at[:, :3].set(W_d2)
    b_d2p = jnp.zeros((1, D), jnp.float32).at[0, :3].set(b_d2)

    def mp(g):
        return mp_call(src, dst, g, zeros_nd)

    h0 = _mm_emb(h, W_emb, b_emb2)
    h1 = _enc(mp(h0), h0, inv, W_enc0, b_enc0_2)
    h2 = _enc(mp(h1), h1, inv, W_enc1, b_enc1_2)

    dt = 1.0 / (T_STEPS - 1)
    sol = [h2]
    hc = h2
    for _ in range(T_STEPS - 1):
        k1, y2 = _rk_stage(0.5 * dt, mp(hc), hc, inv, W_ode, b_ode2, hc)
        k2, y3 = _rk_stage(0.5 * dt, mp(y2), y2, inv, W_ode, b_ode2, hc)
        k3, y4 = _rk_stage(dt, mp(y3), y3, inv, W_ode, b_ode2, hc)
        hc = _rk_final(dt / 6.0, mp(y4), y4, inv, W_ode, b_ode2, hc, k1, k2, k3)
        sol.append(hc)

    hs = jnp.concatenate(sol, axis=0)
    xd = _decode(hs, W_d1, b_d1_2, W_d2p, b_d2p)
    x_final = xd[:, :3]
    dummy_v = jnp.zeros_like(x_final)
    kld = jnp.float32(0.0)
    return (x_final, dummy_v, sol[-1], kld)
